# baseline (speedup 1.0000x reference)
import jax
import jax.numpy as jnp
from jax import lax
from jax.experimental import pallas as pl
from jax.experimental.pallas import tpu as pltpu
from jax.experimental.pallas import tpu_sc as plsc

_HD = 128
_N = 262144
_SC_ROWS = 98304
_TC_HEAD = _N - _SC_ROWS

_NC = 2
_NS = 16
_NW = _NC * _NS
_SC_W = _SC_ROWS // _NW
_CH = 256
_NCH = _SC_W // _CH

_K_GRID = 32
_V_BLK = 4096


def _sc_body(v_hbm, out_hbm, buf0, buf1, si0, si1, so0, so1):
    wid = lax.axis_index("s") * _NC + lax.axis_index("c")
    base = _TC_HEAD + wid * _SC_W
    bufs = (buf0, buf1)
    sin = (si0, si1)
    sout = (so0, so1)

    def make_in(c):
        b = c & 1
        return pltpu.make_async_copy(
            v_hbm.at[pl.ds(base + c * _CH, _CH)], bufs[b], sin[b]
        )

    def make_out(c):
        b = c & 1
        return pltpu.make_async_copy(
            bufs[b], out_hbm.at[pl.ds(base + c * _CH, _CH)], sout[b]
        )

    make_in(0).start()
    for c in range(_NCH):
        make_in(c).wait()
        make_out(c).start()
        if c + 1 < _NCH:
            if c >= 1:
                make_out(c - 1).wait()
            make_in(c + 1).start()
    make_out(_NCH - 2).wait()
    make_out(_NCH - 1).wait()


def _sc_tail_copy(v2):
    mesh = plsc.VectorSubcoreMesh(core_axis_name="c", subcore_axis_name="s")
    f = pl.kernel(
        _sc_body,
        out_type=jax.ShapeDtypeStruct((_N, _HD), v2.dtype),
        mesh=mesh,
        scratch_types=[
            pltpu.VMEM((_CH, _HD), v2.dtype),
            pltpu.VMEM((_CH, _HD), v2.dtype),
            pltpu.SemaphoreType.DMA,
            pltpu.SemaphoreType.DMA,
            pltpu.SemaphoreType.DMA,
            pltpu.SemaphoreType.DMA,
        ],
    )
    return f(v2)


def _tc_copy_body(k_ref, ok_ref):
    ok_ref[...] = k_ref[...]


def _tc_copy(x):
    rows = _N // _K_GRID
    x3 = x.reshape(_K_GRID, rows, _HD)
    spec = pl.BlockSpec((1, rows, _HD), lambda i: (i, 0, 0))
    out = pl.pallas_call(
        _tc_copy_body,
        grid=(_K_GRID,),
        out_shape=jax.ShapeDtypeStruct(x3.shape, x3.dtype),
        in_specs=[spec],
        out_specs=spec,
        compiler_params=pltpu.CompilerParams(
            dimension_semantics=("parallel",),
        ),
    )(x3)
    return out.reshape(_N, _HD)


def _v_head_body(src_ref, _alias_ref, out_ref):
    out_ref[...] = src_ref[...]


def _tc_head_copy(v2, v_partial):
    blk = pl.BlockSpec((_V_BLK, _HD), lambda i: (i, 0))
    return pl.pallas_call(
        _v_head_body,
        grid=(_TC_HEAD // _V_BLK,),
        out_shape=jax.ShapeDtypeStruct((_N, _HD), v2.dtype),
        in_specs=[blk, pl.BlockSpec(memory_space=pl.ANY)],
        out_specs=blk,
        input_output_aliases={1: 0},
        compiler_params=pltpu.CompilerParams(
            dimension_semantics=("arbitrary",),
        ),
    )(v2, v_partial)


def kernel(new_k, new_v):
    B, H, L, HD = new_k.shape
    k2 = new_k.reshape(_N, _HD)
    v2 = new_v.reshape(_N, _HD)
    v_partial = _sc_tail_copy(v2)
    ok = _tc_copy(k2)
    ov = _tc_head_copy(v2, v_partial)
    return ok.reshape(B, H, L, HD), ov.reshape(B, H, L, HD)


# speedup vs baseline: 1.0171x; 1.0171x over previous
import jax
import jax.numpy as jnp
from jax import lax
from jax.experimental import pallas as pl
from jax.experimental.pallas import tpu as pltpu
from jax.experimental.pallas import tpu_sc as plsc

_HD = 128
_N = 262144
_SC_ROWS = 163840
_TC_HEAD = _N - _SC_ROWS

_NC = 2
_NS = 16
_NW = _NC * _NS
_SC_W = _SC_ROWS // _NW
_CH = 256
_NCH = _SC_W // _CH

_K_GRID = 32
_V_BLK = 8192


def _sc_body(v_hbm, out_hbm, buf0, buf1, si0, si1, so0, so1):
    wid = lax.axis_index("s") * _NC + lax.axis_index("c")
    base = _TC_HEAD + wid * _SC_W
    bufs = (buf0, buf1)
    sin = (si0, si1)
    sout = (so0, so1)

    def make_in(c):
        b = c & 1
        return pltpu.make_async_copy(
            v_hbm.at[pl.ds(base + c * _CH, _CH)], bufs[b], sin[b]
        )

    def make_out(c):
        b = c & 1
        return pltpu.make_async_copy(
            bufs[b], out_hbm.at[pl.ds(base + c * _CH, _CH)], sout[b]
        )

    make_in(0).start()
    for c in range(_NCH):
        make_in(c).wait()
        make_out(c).start()
        if c + 1 < _NCH:
            if c >= 1:
                make_out(c - 1).wait()
            make_in(c + 1).start()
    make_out(_NCH - 2).wait()
    make_out(_NCH - 1).wait()


def _sc_tail_copy(v2):
    mesh = plsc.VectorSubcoreMesh(core_axis_name="c", subcore_axis_name="s")
    f = pl.kernel(
        _sc_body,
        out_type=jax.ShapeDtypeStruct((_N, _HD), v2.dtype),
        mesh=mesh,
        scratch_types=[
            pltpu.VMEM((_CH, _HD), v2.dtype),
            pltpu.VMEM((_CH, _HD), v2.dtype),
            pltpu.SemaphoreType.DMA,
            pltpu.SemaphoreType.DMA,
            pltpu.SemaphoreType.DMA,
            pltpu.SemaphoreType.DMA,
        ],
    )
    return f(v2)


def _tc_copy_body(k_ref, ok_ref):
    ok_ref[...] = k_ref[...]


def _tc_copy(x):
    rows = _N // _K_GRID
    x3 = x.reshape(_K_GRID, rows, _HD)
    spec = pl.BlockSpec((1, rows, _HD), lambda i: (i, 0, 0))
    out = pl.pallas_call(
        _tc_copy_body,
        grid=(_K_GRID,),
        out_shape=jax.ShapeDtypeStruct(x3.shape, x3.dtype),
        in_specs=[spec],
        out_specs=spec,
        compiler_params=pltpu.CompilerParams(
            dimension_semantics=("parallel",),
        ),
    )(x3)
    return out.reshape(_N, _HD)


def _v_head_body(src_ref, _alias_ref, out_ref):
    out_ref[...] = src_ref[...]


def _tc_head_copy(v2, v_partial):
    blk = pl.BlockSpec((_V_BLK, _HD), lambda i: (i, 0))
    return pl.pallas_call(
        _v_head_body,
        grid=(_TC_HEAD // _V_BLK,),
        out_shape=jax.ShapeDtypeStruct((_N, _HD), v2.dtype),
        in_specs=[blk, pl.BlockSpec(memory_space=pl.ANY)],
        out_specs=blk,
        input_output_aliases={1: 0},
        compiler_params=pltpu.CompilerParams(
            dimension_semantics=("arbitrary",),
        ),
    )(v2, v_partial)


def kernel(new_k, new_v):
    B, H, L, HD = new_k.shape
    k2 = new_k.reshape(_N, _HD)
    v2 = new_v.reshape(_N, _HD)
    v_partial = _sc_tail_copy(v2)
    ok = _tc_copy(k2)
    ov = _tc_head_copy(v2, v_partial)
    return ok.reshape(B, H, L, HD), ov.reshape(B, H, L, HD)


# speedup vs baseline: 1.1549x; 1.1355x over previous
import jax
import jax.numpy as jnp
from jax.experimental import pallas as pl
from jax.experimental.pallas import tpu as pltpu

_GRID = 32


def _copy_body(k_ref, v_ref, ok_ref, ov_ref):
    ok_ref[...] = k_ref[...]
    ov_ref[...] = v_ref[...]


def kernel(new_k, new_v):
    B, H, L, HD = new_k.shape
    rows = B * H * L // _GRID
    k2 = new_k.reshape(_GRID, rows, HD)
    v2 = new_v.reshape(_GRID, rows, HD)
    out_shape = (
        jax.ShapeDtypeStruct(k2.shape, k2.dtype),
        jax.ShapeDtypeStruct(v2.shape, v2.dtype),
    )
    spec = pl.BlockSpec((1, rows, HD), lambda i: (i, 0, 0))
    ok, ov = pl.pallas_call(
        _copy_body,
        grid=(_GRID,),
        out_shape=out_shape,
        in_specs=[spec, spec],
        out_specs=[spec, spec],
        compiler_params=pltpu.CompilerParams(
            dimension_semantics=("arbitrary",),
            disable_bounds_checks=True,
            disable_semaphore_checks=True,
            skip_device_barrier=True,
        ),
    )(k2, v2)
    return ok.reshape(B, H, L, HD), ov.reshape(B, H, L, HD)
